# Initial kernel scaffold; baseline (speedup 1.0000x reference)
import jax
import jax.numpy as jnp
from jax.experimental import pallas as pl


def kernel(x, rbf, sbf, W_rbf1, W_rbf2, W_sbf1, W_sbf2, W_kj, b_kj, W_ji, b_ji, W_down, W_up, Wb1, bb1, Wb2, bb2, W_lin, b_lin, Wa1, ba1, Wa2, ba2, idx_kj, idx_ji):
    raise NotImplementedError("write your pallas kernel here")


# speedup vs baseline: 2.4293x; 2.4293x over previous
import functools

import jax
import jax.numpy as jnp
from jax import lax
from jax.experimental import pallas as pl
from jax.experimental.pallas import tpu as pltpu
from jax.experimental.pallas import tpu_sc as plsc

E = 160000
T = 640000
H = 128
INTD = 64

NC = 2
NS = 16
L = 16

CH = 13504
NCH = 12
EPAD = CH * NCH
SHARE = 40960
B = 512
KB = B // 128
STRIPE = CH // NS


def _silu(v):
    return v / (1.0 + jnp.exp(-v))


def _pre_body(x_ref, rbf_ref, wkj_ref, bkj_ref, wji_ref, bji_ref,
              wr1_ref, wr2_ref, wd_ref, xji_ref, down_ref):
    xb = x_ref[...]
    xji_ref[...] = _silu(xb @ wji_ref[...] + bji_ref[...])
    xkj = _silu(xb @ wkj_ref[...] + bkj_ref[...])
    rbf_e = (rbf_ref[...] @ wr1_ref[...]) @ wr2_ref[...]
    down_ref[...] = _silu((xkj * rbf_e) @ wd_ref[...])


def _pre_call(x, rbf8, wkj, bkj, wji, bji, wr1, wr2, wd):
    be = 2000
    grid = (E // be,)
    full = lambda a: pl.BlockSpec(a.shape, lambda i: (0,) * a.ndim)
    return pl.pallas_call(
        _pre_body,
        grid=grid,
        in_specs=[
            pl.BlockSpec((be, H), lambda i: (i, 0)),
            pl.BlockSpec((be, 8), lambda i: (i, 0)),
            full(wkj), full(bkj), full(wji), full(bji),
            full(wr1), full(wr2), full(wd),
        ],
        out_specs=[
            pl.BlockSpec((be, H), lambda i: (i, 0)),
            pl.BlockSpec((be, INTD), lambda i: (i, 0)),
        ],
        out_shape=[
            jax.ShapeDtypeStruct((E, H), jnp.float32),
            jax.ShapeDtypeStruct((E, INTD), jnp.float32),
        ],
        compiler_params=pltpu.CompilerParams(dimension_semantics=("arbitrary",)),
    )(x, rbf8, wkj, bkj, wji, bji, wr1, wr2, wd)


def _sbf_body(sbf_ref, w1_ref, w2_ref, out_ref):
    out_ref[...] = (sbf_ref[...] @ w1_ref[...]) @ w2_ref[...]


def _sbf_call(sbf, w1, w2):
    bt = 4000
    grid = (T // bt,)
    full = lambda a: pl.BlockSpec(a.shape, lambda i: (0,) * a.ndim)
    return pl.pallas_call(
        _sbf_body,
        grid=grid,
        in_specs=[
            pl.BlockSpec((bt, sbf.shape[1]), lambda i: (i, 0)),
            full(w1), full(w2),
        ],
        out_specs=pl.BlockSpec((bt, INTD), lambda i: (i, 0)),
        out_shape=jax.ShapeDtypeStruct((T, INTD), jnp.float32),
        compiler_params=pltpu.CompilerParams(dimension_semantics=("arbitrary",)),
    )(sbf, w1, w2)


def _post_body(seg_ref, xji_ref, x_ref, wup_ref, wb1_ref, bb1_ref, wb2_ref,
               bb2_ref, wlin_ref, blin_ref, wa1_ref, ba1_ref, wa2_ref,
               ba2_ref, out_ref):
    h = xji_ref[...] + _silu(seg_ref[...] @ wup_ref[...])
    h = h + _silu(_silu(h @ wb1_ref[...] + bb1_ref[...]) @ wb2_ref[...]
                  + bb2_ref[...])
    h = _silu(h @ wlin_ref[...] + blin_ref[...]) + x_ref[...]
    h = h + _silu(_silu(h @ wa1_ref[...] + ba1_ref[...]) @ wa2_ref[...]
                  + ba2_ref[...])
    out_ref[...] = h


def _post_call(seg, xji, x, wup, wb1, bb1, wb2, bb2, wlin, blin,
               wa1, ba1, wa2, ba2):
    be = 2000
    grid = (E // be,)
    full = lambda a: pl.BlockSpec(a.shape, lambda i: (0,) * a.ndim)
    return pl.pallas_call(
        _post_body,
        grid=grid,
        in_specs=[
            pl.BlockSpec((be, INTD), lambda i: (i, 0)),
            pl.BlockSpec((be, H), lambda i: (i, 0)),
            pl.BlockSpec((be, H), lambda i: (i, 0)),
            full(wup), full(wb1), full(bb1), full(wb2), full(bb2),
            full(wlin), full(blin), full(wa1), full(ba1), full(wa2), full(ba2),
        ],
        out_specs=pl.BlockSpec((be, H), lambda i: (i, 0)),
        out_shape=jax.ShapeDtypeStruct((E, H), jnp.float32),
        compiler_params=pltpu.CompilerParams(dimension_semantics=("arbitrary",)),
    )(seg, xji, x, wup, wb1, bb1, wb2, bb2, wlin, blin, wa1, ba1, wa2, ba2)


def _sc_body(down_hbm, sbfe_hbm, kj_hbm, ji_hbm, out_hbm,
             ji2d, kj2d, loc2d, rows, srows, spmem, sem):
    cid = lax.axis_index("c")
    sid = lax.axis_index("s")
    s0 = sid * SHARE
    nb = (jnp.minimum(SHARE, T - s0)) // B

    for ch in range(NCH // NC):
        chunk = cid * (NCH // NC) + ch
        lo = chunk * CH
        trash = CH + sid

        @pl.loop(0, B)
        def _zero(r):
            for c in range(INTD // L):
                rows[r, pl.ds(c * L, L)] = jnp.zeros((L,), jnp.float32)

        r0 = sid * STRIPE
        off = 0
        while off < STRIPE:
            sz = min(B, STRIPE - off)
            pltpu.sync_copy(rows.at[pl.ds(0, sz)],
                            spmem.at[pl.ds(r0 + off, sz)])
            off += sz
        pltpu.sync_copy(rows.at[pl.ds(0, 1)], spmem.at[pl.ds(trash, 1)])
        plsc.subcore_barrier()

        @pl.loop(0, nb)
        def _block(blk):
            t0 = s0 + blk * B
            descs = []
            for j in range(KB):
                descs.append(pltpu.async_copy(
                    ji_hbm.at[pl.ds(t0 + j * 128, 128)], ji2d.at[j], sem))
                descs.append(pltpu.async_copy(
                    kj_hbm.at[pl.ds(t0 + j * 128, 128)], kj2d.at[j], sem))
            descs.append(pltpu.async_copy(
                sbfe_hbm.at[pl.ds(t0, B)], srows, sem))
            for d in descs:
                d.wait()

            for j in range(KB):
                for k in range(128 // L):
                    jiv = ji2d[j, pl.ds(k * L, L)]
                    ok = (jiv >= lo) & (jiv < lo + CH)
                    loc2d[j, pl.ds(k * L, L)] = jnp.where(ok, jiv - lo, trash)

            descs = []
            for j in range(KB):
                descs.append(pltpu.async_copy(
                    down_hbm.at[kj2d.at[j]],
                    rows.at[pl.ds(j * 128, 128)], sem))
            for d in descs:
                d.wait()

            @pl.loop(0, B)
            def _mul(r):
                for c in range(INTD // L):
                    sl = pl.ds(c * L, L)
                    rows[r, sl] = rows[r, sl] * srows[r, sl]

            for j in range(KB):
                pltpu.sync_copy(rows.at[pl.ds(j * 128, 128)],
                                spmem.at[loc2d.at[j]], add=True)

        plsc.subcore_barrier()

        off = 0
        while off < STRIPE:
            sz = min(B, STRIPE - off)
            pltpu.sync_copy(spmem.at[pl.ds(r0 + off, sz)],
                            out_hbm.at[pl.ds(lo + r0 + off, sz)])
            off += sz
        plsc.subcore_barrier()


def _sc_segment(down, sbfe, idx_kj, idx_ji):
    mesh = plsc.VectorSubcoreMesh(core_axis_name="c", subcore_axis_name="s",
                                  num_cores=NC, num_subcores=NS)
    k = pl.kernel(
        _sc_body,
        out_type=jax.ShapeDtypeStruct((EPAD, INTD), jnp.float32),
        mesh=mesh,
        scratch_types=[
            pltpu.VMEM((KB, 128), jnp.int32),
            pltpu.VMEM((KB, 128), jnp.int32),
            pltpu.VMEM((KB, 128), jnp.int32),
            pltpu.VMEM((B, INTD), jnp.float32),
            pltpu.VMEM((B, INTD), jnp.float32),
            pltpu.VMEM_SHARED((CH + NS, INTD), jnp.float32),
            pltpu.SemaphoreType.DMA,
        ],
        compiler_params=pltpu.CompilerParams(use_tc_tiling_on_sc=False),
    )
    return k(down, sbfe, idx_kj.astype(jnp.int32), idx_ji.astype(jnp.int32))


def kernel(x, rbf, sbf, W_rbf1, W_rbf2, W_sbf1, W_sbf2, W_kj, b_kj, W_ji,
           b_ji, W_down, W_up, Wb1, bb1, Wb2, bb2, W_lin, b_lin, Wa1, ba1,
           Wa2, ba2, idx_kj, idx_ji):
    rbf8 = jnp.pad(rbf, ((0, 0), (0, 2)))
    wr18 = jnp.pad(W_rbf1, ((0, 2), (0, 0)))
    b2 = lambda b: b.reshape(1, -1)

    xji, down = _pre_call(x, rbf8, W_kj, b2(b_kj), W_ji, b2(b_ji),
                          wr18, W_rbf2, W_down)
    sbfe = _sbf_call(sbf, W_sbf1, W_sbf2)
    seg = _sc_segment(down, sbfe, idx_kj, idx_ji)[:E]
    return _post_call(seg, xji, x, W_up, Wb1, b2(bb1), Wb2, b2(bb2),
                      W_lin, b2(b_lin), Wa1, b2(ba1), Wa2, b2(ba2))
